# baseline (speedup 1.0000x reference)
import functools

import jax
import jax.numpy as jnp
from jax import lax
from jax.experimental import pallas as pl
from jax.experimental.pallas import tpu as pltpu
from jax.experimental.pallas import tpu_sc as plsc

VOCAB = 100000
BOTTLENECK = 128
D_MODEL = 1024
BATCH = 4
SEQ = 2048
N_TOKENS = BATCH * SEQ

NUM_CORES = 2
NUM_SUBCORES = 16
NW = NUM_CORES * NUM_SUBCORES
B_PER_W = N_TOKENS // NW
W_PER_ROW = SEQ // B_PER_W
CHUNK = 32
NCHUNK = B_PER_W // CHUNK

_sc_mesh = plsc.VectorSubcoreMesh(core_axis_name="c", subcore_axis_name="s")


@functools.partial(
    pl.kernel,
    mesh=_sc_mesh,
    out_type=jax.ShapeDtypeStruct((N_TOKENS, BOTTLENECK), jnp.float32),
    scratch_types=[
        pltpu.VMEM((B_PER_W,), jnp.int32),
        pltpu.VMEM((B_PER_W, BOTTLENECK), jnp.float32),
    ]
    + [pltpu.SemaphoreType.DMA] * NCHUNK
    + [pltpu.SemaphoreType.DMA],
)
def _sc_gather(table_hbm, idx_hbm, out_hbm, idx_v, rows_v, *sems):
    gsems, wsem = sems[:NCHUNK], sems[NCHUNK]
    wid = lax.axis_index("s") * NUM_CORES + lax.axis_index("c")
    row = wid // W_PER_ROW
    col = (wid % W_PER_ROW) * B_PER_W
    base = wid * B_PER_W
    pltpu.sync_copy(idx_hbm.at[row, pl.ds(col, B_PER_W)], idx_v)
    gathers = []
    for j in range(NCHUNK):
        gathers.append(
            pltpu.async_copy(
                table_hbm.at[idx_v.at[pl.ds(j * CHUNK, CHUNK)]],
                rows_v.at[pl.ds(j * CHUNK, CHUNK)],
                gsems[j],
            )
        )
    writebacks = []
    for j in range(NCHUNK):
        gathers[j].wait()
        writebacks.append(
            pltpu.async_copy(
                rows_v.at[pl.ds(j * CHUNK, CHUNK)],
                out_hbm.at[pl.ds(base + j * CHUNK, CHUNK)],
                wsem,
            )
        )
    for wb in writebacks:
        wb.wait()


def _mm_body(low_ref, w_ref, out_ref):
    out_ref[...] = jnp.dot(
        low_ref[...], w_ref[...], preferred_element_type=jnp.float32
    )


ROW_TILE = 2048


@jax.jit
def kernel(x, embed_table, W):
    idx = x.astype(jnp.int32)
    low = _sc_gather(embed_table, idx)
    out = pl.pallas_call(
        _mm_body,
        grid=(N_TOKENS // ROW_TILE,),
        in_specs=[
            pl.BlockSpec((ROW_TILE, BOTTLENECK), lambda i: (i, 0)),
            pl.BlockSpec((BOTTLENECK, D_MODEL), lambda i: (0, 0)),
        ],
        out_specs=pl.BlockSpec((ROW_TILE, D_MODEL), lambda i: (i, 0)),
        out_shape=jax.ShapeDtypeStruct((N_TOKENS, D_MODEL), jnp.float32),
    )(low, W)
    return out.reshape(x.shape[0], x.shape[1], D_MODEL)


# speedup vs baseline: 1.0042x; 1.0042x over previous
import functools

import jax
import jax.numpy as jnp
from jax import lax
from jax.experimental import pallas as pl
from jax.experimental.pallas import tpu as pltpu
from jax.experimental.pallas import tpu_sc as plsc

VOCAB = 100000
BOTTLENECK = 128
D_MODEL = 1024
BATCH = 4
SEQ = 2048
N_TOKENS = BATCH * SEQ

NUM_CORES = 2
NUM_SUBCORES = 16
NW = NUM_CORES * NUM_SUBCORES
B_PER_W = N_TOKENS // NW
W_PER_ROW = SEQ // B_PER_W
CHUNK = 64
NCHUNK = B_PER_W // CHUNK

_sc_mesh = plsc.VectorSubcoreMesh(core_axis_name="c", subcore_axis_name="s")


@functools.partial(
    pl.kernel,
    mesh=_sc_mesh,
    out_type=jax.ShapeDtypeStruct((N_TOKENS, BOTTLENECK), jnp.float32),
    scratch_types=[
        pltpu.VMEM((B_PER_W,), jnp.int32),
        pltpu.VMEM((B_PER_W, BOTTLENECK), jnp.float32),
    ]
    + [pltpu.SemaphoreType.DMA] * NCHUNK
    + [pltpu.SemaphoreType.DMA],
)
def _sc_gather(table_hbm, idx_hbm, out_hbm, idx_v, rows_v, *sems):
    gsems, wsem = sems[:NCHUNK], sems[NCHUNK]
    wid = lax.axis_index("s") * NUM_CORES + lax.axis_index("c")
    row = wid // W_PER_ROW
    col = (wid % W_PER_ROW) * B_PER_W
    base = wid * B_PER_W
    pltpu.sync_copy(idx_hbm.at[row, pl.ds(col, B_PER_W)], idx_v)
    gathers = []
    for j in range(NCHUNK):
        gathers.append(
            pltpu.async_copy(
                table_hbm.at[idx_v.at[pl.ds(j * CHUNK, CHUNK)]],
                rows_v.at[pl.ds(j * CHUNK, CHUNK)],
                gsems[j],
            )
        )
    writebacks = []
    for j in range(NCHUNK):
        gathers[j].wait()
        writebacks.append(
            pltpu.async_copy(
                rows_v.at[pl.ds(j * CHUNK, CHUNK)],
                out_hbm.at[pl.ds(base + j * CHUNK, CHUNK)],
                wsem,
            )
        )
    for wb in writebacks:
        wb.wait()


def _mm_body(low_ref, w_ref, out_ref):
    out_ref[...] = jnp.dot(
        low_ref[...], w_ref[...], preferred_element_type=jnp.float32
    )


ROW_TILE = 2048


@jax.jit
def kernel(x, embed_table, W):
    idx = x.astype(jnp.int32)
    low = _sc_gather(embed_table, idx)
    out = pl.pallas_call(
        _mm_body,
        grid=(N_TOKENS // ROW_TILE,),
        in_specs=[
            pl.BlockSpec((ROW_TILE, BOTTLENECK), lambda i: (i, 0)),
            pl.BlockSpec((BOTTLENECK, D_MODEL), lambda i: (0, 0)),
        ],
        out_specs=pl.BlockSpec((ROW_TILE, D_MODEL), lambda i: (i, 0)),
        out_shape=jax.ShapeDtypeStruct((N_TOKENS, D_MODEL), jnp.float32),
    )(low, W)
    return out.reshape(x.shape[0], x.shape[1], D_MODEL)


# speedup vs baseline: 1.0416x; 1.0372x over previous
import functools

import jax
import jax.numpy as jnp
from jax import lax
from jax.experimental import pallas as pl
from jax.experimental.pallas import tpu as pltpu
from jax.experimental.pallas import tpu_sc as plsc

VOCAB = 100000
BOTTLENECK = 128
D_MODEL = 1024
BATCH = 4
SEQ = 2048
N_TOKENS = BATCH * SEQ

NUM_CORES = 2
NUM_SUBCORES = 16
NW = NUM_CORES * NUM_SUBCORES
B_PER_W = N_TOKENS // NW
W_PER_ROW = SEQ // B_PER_W
CHUNK = 128
NCHUNK = B_PER_W // CHUNK

_sc_mesh = plsc.VectorSubcoreMesh(core_axis_name="c", subcore_axis_name="s")


@functools.partial(
    pl.kernel,
    mesh=_sc_mesh,
    out_type=jax.ShapeDtypeStruct((N_TOKENS, BOTTLENECK), jnp.float32),
    scratch_types=[
        pltpu.VMEM((B_PER_W,), jnp.int32),
        pltpu.VMEM((B_PER_W, BOTTLENECK), jnp.float32),
    ]
    + [pltpu.SemaphoreType.DMA] * NCHUNK
    + [pltpu.SemaphoreType.DMA],
)
def _sc_gather(table_hbm, idx_hbm, out_hbm, idx_v, rows_v, *sems):
    gsems, wsem = sems[:NCHUNK], sems[NCHUNK]
    wid = lax.axis_index("s") * NUM_CORES + lax.axis_index("c")
    row = wid // W_PER_ROW
    col = (wid % W_PER_ROW) * B_PER_W
    base = wid * B_PER_W
    pltpu.sync_copy(idx_hbm.at[row, pl.ds(col, B_PER_W)], idx_v)
    gathers = []
    for j in range(NCHUNK):
        gathers.append(
            pltpu.async_copy(
                table_hbm.at[idx_v.at[pl.ds(j * CHUNK, CHUNK)]],
                rows_v.at[pl.ds(j * CHUNK, CHUNK)],
                gsems[j],
            )
        )
    writebacks = []
    for j in range(NCHUNK):
        gathers[j].wait()
        writebacks.append(
            pltpu.async_copy(
                rows_v.at[pl.ds(j * CHUNK, CHUNK)],
                out_hbm.at[pl.ds(base + j * CHUNK, CHUNK)],
                wsem,
            )
        )
    for wb in writebacks:
        wb.wait()


def _mm_body(low_ref, w_ref, out_ref):
    out_ref[...] = jnp.dot(
        low_ref[...], w_ref[...], preferred_element_type=jnp.float32
    )


ROW_TILE = 2048


@jax.jit
def kernel(x, embed_table, W):
    idx = x.astype(jnp.int32)
    low = _sc_gather(embed_table, idx)
    out = pl.pallas_call(
        _mm_body,
        grid=(N_TOKENS // ROW_TILE,),
        in_specs=[
            pl.BlockSpec((ROW_TILE, BOTTLENECK), lambda i: (i, 0)),
            pl.BlockSpec((BOTTLENECK, D_MODEL), lambda i: (0, 0)),
        ],
        out_specs=pl.BlockSpec((ROW_TILE, D_MODEL), lambda i: (i, 0)),
        out_shape=jax.ShapeDtypeStruct((N_TOKENS, D_MODEL), jnp.float32),
    )(low, W)
    return out.reshape(x.shape[0], x.shape[1], D_MODEL)
